# baseline (speedup 1.0000x reference)
import functools

import jax
import jax.numpy as jnp
from jax.experimental import pallas as pl

TOKENS = 32768
DIM = 768
N_EXPERTS = 8
LANES = 128
BLOCK = 2048
STREAMS = 2

NEG_INF = float("-inf")


def _top1(s):
    lane = jax.lax.broadcasted_iota(jnp.int32, s.shape, 1)
    s = jnp.where(lane < N_EXPERTS, s, NEG_INF)
    m = jnp.max(s, axis=1, keepdims=True)
    denom = jnp.sum(jnp.exp(s - m), axis=1, keepdims=True)
    return 1.0 / denom, jnp.argmax(s, axis=1).reshape(-1, 1).astype(jnp.int32)


def _gate_kernel(*refs):
    x_refs = refs[:STREAMS]
    wt_ref = refs[STREAMS]
    w_out_ref, idx_out_ref = refs[STREAMS + 1:]
    wt = wt_ref[...]
    for j in range(STREAMS):
        s = jnp.dot(x_refs[j][...], wt, preferred_element_type=jnp.float32)
        w, idx = _top1(s)
        w_out_ref[pl.ds(j * BLOCK, BLOCK), :] = w
        idx_out_ref[pl.ds(j * BLOCK, BLOCK), :] = idx


@jax.jit
def kernel(x, weight):
    wt = jnp.zeros((DIM, LANES), dtype=jnp.float32).at[:, :N_EXPERTS].set(
        weight.T)
    grid = (TOKENS // (BLOCK * STREAMS),)
    in_specs = [
        pl.BlockSpec((BLOCK, DIM), functools.partial(
            lambda j, i: (i * STREAMS + j, 0), j))
        for j in range(STREAMS)
    ]
    in_specs.append(pl.BlockSpec((DIM, LANES), lambda i: (0, 0)))
    weights, indices = pl.pallas_call(
        _gate_kernel,
        grid=grid,
        in_specs=in_specs,
        out_specs=[
            pl.BlockSpec((BLOCK * STREAMS, 1), lambda i: (i, 0)),
            pl.BlockSpec((BLOCK * STREAMS, 1), lambda i: (i, 0)),
        ],
        out_shape=[
            jax.ShapeDtypeStruct((TOKENS, 1), jnp.float32),
            jax.ShapeDtypeStruct((TOKENS, 1), jnp.int32),
        ],
    )(*([x] * STREAMS), wt)
    return weights, indices
